# Initial kernel scaffold; baseline (speedup 1.0000x reference)
import jax
import jax.numpy as jnp
from jax.experimental import pallas as pl


def kernel(x, edge_index, W_rel1, W_root1, b1, W_rel2, W_root2, b2, W_rel3, W_root3, b3, lin_W, lin_b):
    raise NotImplementedError("write your pallas kernel here")


# speedup vs baseline: 3.8717x; 3.8717x over previous
import functools

import jax
import jax.numpy as jnp
from jax import lax
from jax.experimental import pallas as pl
from jax.experimental.pallas import tpu as pltpu
from jax.experimental.pallas import tpu_sc as plsc

N = 10000
F = 128
H = 20
OUT = 128
NP = 10240
HP = 128
NC = 2
NS = 16
CHUNK = 128
GROUP = 16
E = 320000
EDGES_PER_W = 10240
NCHUNK = EDGES_PER_W // CHUNK
NGROUP = NCHUNK // GROUP
EP = EDGES_PER_W * NC * NS
ROWS_PER_S = NP // NS


def _sc_segment_sum(p, src3, dst3, zeros):
    mesh = plsc.VectorSubcoreMesh(core_axis_name="c", subcore_axis_name="s")

    @functools.partial(
        pl.kernel,
        out_type=jax.ShapeDtypeStruct((NC, NP, HP), jnp.float32),
        mesh=mesh,
        scratch_types=[
            pltpu.VMEM((GROUP, CHUNK), jnp.int32),
            pltpu.VMEM((GROUP, CHUNK), jnp.int32),
            pltpu.VMEM((CHUNK, HP), jnp.float32),
            pltpu.VMEM((CHUNK, HP), jnp.float32),
            pltpu.VMEM_SHARED((NP, HP), jnp.float32),
            pltpu.SemaphoreType.DMA,
            pltpu.SemaphoreType.DMA,
        ],
    )
    def k(p_hbm, src_hbm, dst_hbm, z_hbm, out_hbm,
          src_v, dst_v, buf_a, buf_b, acc, sem_a, sem_b):
        c = lax.axis_index("c")
        s = lax.axis_index("s")
        wid = c * NS + s

        pltpu.sync_copy(z_hbm.at[pl.ds(s * ROWS_PER_S, ROWS_PER_S)],
                        acc.at[pl.ds(s * ROWS_PER_S, ROWS_PER_S)])
        plsc.subcore_barrier()

        def gather(j, buf, sem):
            pltpu.async_copy(p_hbm.at[src_v.at[j]], buf, sem)

        def drain_add(j, buf, sem):
            pltpu.make_async_copy(p_hbm.at[src_v.at[j]], buf, sem).wait()
            pltpu.sync_copy(buf, acc.at[dst_v.at[j]], add=True)

        bufs = ((buf_a, sem_a), (buf_b, sem_b))
        for g in range(NGROUP):
            pltpu.sync_copy(src_hbm.at[wid, pl.ds(g * GROUP, GROUP)], src_v)
            pltpu.sync_copy(dst_hbm.at[wid, pl.ds(g * GROUP, GROUP)], dst_v)
            gather(0, *bufs[0])
            for k in range(GROUP - 1):
                gather(k + 1, *bufs[(k + 1) % 2])
                drain_add(k, *bufs[k % 2])
            drain_add(GROUP - 1, *bufs[(GROUP - 1) % 2])

        plsc.subcore_barrier()
        pltpu.sync_copy(acc.at[pl.ds(s * ROWS_PER_S, ROWS_PER_S)],
                        out_hbm.at[c].at[pl.ds(s * ROWS_PER_S, ROWS_PER_S)])

    return k(p, src3, dst3, zeros)


def _proj1_kernel(x_ref, w_ref, o_ref):
    o_ref[0:N, :] = jnp.dot(x_ref[...], w_ref[...].T,
                            preferred_element_type=jnp.float32)
    o_ref[N:NP, :] = jnp.zeros((NP - N, HP), jnp.float32)


def _proj1(x, w_rel1p):
    return pl.pallas_call(
        _proj1_kernel,
        out_shape=jax.ShapeDtypeStruct((NP, HP), jnp.float32),
    )(x, w_rel1p)


def _combine_kernel(nrows, parts_ref, hprev_ref, wroot_ref, b_ref, wnext_ref,
                    h_ref, p_ref):
    root = jnp.dot(hprev_ref[...], wroot_ref[...].T,
                   preferred_element_type=jnp.float32)
    agg = parts_ref[0, 0:nrows, :] + parts_ref[1, 0:nrows, :]
    h_ref[0:nrows, :] = jax.nn.relu(agg + root + b_ref[...])
    if nrows < NP:
        h_ref[nrows:NP, :] = jnp.zeros((NP - nrows, HP), jnp.float32)
    p_ref[...] = jnp.dot(h_ref[...], wnext_ref[...].T,
                         preferred_element_type=jnp.float32)


def _combine(parts, hprev, wrootp, bp, wnextp):
    nrows = hprev.shape[0]
    return pl.pallas_call(
        functools.partial(_combine_kernel, nrows),
        out_shape=(jax.ShapeDtypeStruct((NP, HP), jnp.float32),
                   jax.ShapeDtypeStruct((NP, HP), jnp.float32)),
    )(parts, hprev, wrootp, bp, wnextp)


def _final_kernel(parts_ref, h2_ref, wroot_ref, b_ref, linw_ref, linb_ref,
                  o_ref):
    root = jnp.dot(h2_ref[...], wroot_ref[...].T,
                   preferred_element_type=jnp.float32)
    h3 = jax.nn.relu(parts_ref[0] + parts_ref[1] + root + b_ref[...])
    hv = h3[0:N, :]
    max_p = jnp.max(hv, axis=0)
    mean_p = jnp.sum(hv, axis=0) * (1.0 / N)
    pooled = jnp.concatenate([max_p, mean_p])[None, :]
    o_ref[...] = jnp.dot(pooled, linw_ref[...].T,
                         preferred_element_type=jnp.float32) + linb_ref[...]


def _final(parts, h2, wroot3p, b3p, lin_wp, lin_bp):
    return pl.pallas_call(
        _final_kernel,
        out_shape=jax.ShapeDtypeStruct((1, OUT), jnp.float32),
    )(parts, h2, wroot3p, b3p, lin_wp, lin_bp)


def _pad_w(w):
    return jnp.pad(w, ((0, HP - H), (0, 0)))


def _pad_w_sq(w):
    return jnp.pad(w, ((0, HP - H), (0, HP - H)))


def kernel(x, edge_index, W_rel1, W_root1, b1, W_rel2, W_root2, b2,
           W_rel3, W_root3, b3, lin_W, lin_b):
    w_rel1p = _pad_w(W_rel1)
    w_root1p = _pad_w(W_root1)
    w_rel2p = _pad_w_sq(W_rel2)
    w_root2p = _pad_w_sq(W_root2)
    w_rel3p = _pad_w_sq(W_rel3)
    w_root3p = _pad_w_sq(W_root3)
    b1p = jnp.pad(b1, (0, HP - H))[None, :]
    b2p = jnp.pad(b2, (0, HP - H))[None, :]
    b3p = jnp.pad(b3, (0, HP - H))[None, :]
    lin_wp = jnp.zeros((OUT, 2 * HP), jnp.float32)
    lin_wp = lin_wp.at[:, 0:H].set(lin_W[:, 0:H])
    lin_wp = lin_wp.at[:, HP:HP + H].set(lin_W[:, H:2 * H])
    lin_bp = lin_b[None, :]

    pad_e = EP - E
    src = jnp.pad(edge_index[0], (0, pad_e), constant_values=NP - 1)
    dst = jnp.pad(edge_index[1], (0, pad_e), constant_values=NP - 1)
    src3 = src.reshape(NC * NS, NCHUNK, CHUNK)
    dst3 = dst.reshape(NC * NS, NCHUNK, CHUNK)
    zeros = jnp.zeros((NP, HP), jnp.float32)

    p1 = _proj1(x, w_rel1p)
    parts1 = _sc_segment_sum(p1, src3, dst3, zeros)
    h1, p2 = _combine(parts1, x, w_root1p, b1p, w_rel2p)
    parts2 = _sc_segment_sum(p2, src3, dst3, zeros)
    h2, p3 = _combine(parts2, h1, w_root2p, b2p, w_rel3p)
    parts3 = _sc_segment_sum(p3, src3, dst3, zeros)
    return _final(parts3, h2, w_root3p, b3p, lin_wp, lin_bp)


# speedup vs baseline: 4.2103x; 1.0874x over previous
import functools

import jax
import jax.numpy as jnp
from jax import lax
from jax.experimental import pallas as pl
from jax.experimental.pallas import tpu as pltpu
from jax.experimental.pallas import tpu_sc as plsc

N = 10000
F = 128
H = 20
OUT = 128
NP = 10240
HP = 128
NC = 2
NS = 16
CHUNK = 128
GROUP = 16
E = 320000
G0 = 8
G1 = 2
NCHUNK = G0 * GROUP
E0 = NS * G0 * GROUP * CHUNK
E1 = NS * G1 * GROUP * CHUNK
ROWS_PER_S = NP // NS


def _sc_segment_sum(p, src3, dst3, zeros):
    mesh = plsc.VectorSubcoreMesh(core_axis_name="c", subcore_axis_name="s")

    @functools.partial(
        pl.kernel,
        out_type=jax.ShapeDtypeStruct((NC, NP, HP), jnp.float32),
        mesh=mesh,
        scratch_types=[
            pltpu.VMEM((GROUP, CHUNK), jnp.int32),
            pltpu.VMEM((GROUP, CHUNK), jnp.int32),
            pltpu.VMEM((CHUNK, HP), jnp.float32),
            pltpu.VMEM((CHUNK, HP), jnp.float32),
            pltpu.VMEM_SHARED((NP, HP), jnp.float32),
            pltpu.SemaphoreType.DMA,
            pltpu.SemaphoreType.DMA,
        ],
    )
    def k(p_hbm, src_hbm, dst_hbm, z_hbm, out_hbm,
          src_v, dst_v, buf_a, buf_b, acc, sem_a, sem_b):
        c = lax.axis_index("c")
        s = lax.axis_index("s")
        wid = c * NS + s

        pltpu.sync_copy(z_hbm.at[pl.ds(s * ROWS_PER_S, ROWS_PER_S)],
                        acc.at[pl.ds(s * ROWS_PER_S, ROWS_PER_S)])
        plsc.subcore_barrier()

        def gather(j, buf, sem):
            pltpu.async_copy(p_hbm.at[src_v.at[j]], buf, sem)

        def drain_add(j, buf, sem):
            pltpu.make_async_copy(p_hbm.at[src_v.at[j]], buf, sem).wait()
            pltpu.sync_copy(buf, acc.at[dst_v.at[j]], add=True)

        bufs = ((buf_a, sem_a), (buf_b, sem_b))
        ngroups = jnp.where(c == 0, G0, G1)

        @pl.loop(0, ngroups)
        def _(g):
            pltpu.sync_copy(src_hbm.at[wid, pl.ds(g * GROUP, GROUP)], src_v)
            pltpu.sync_copy(dst_hbm.at[wid, pl.ds(g * GROUP, GROUP)], dst_v)
            gather(0, *bufs[0])
            for k in range(GROUP - 1):
                gather(k + 1, *bufs[(k + 1) % 2])
                drain_add(k, *bufs[k % 2])
            drain_add(GROUP - 1, *bufs[(GROUP - 1) % 2])

        plsc.subcore_barrier()
        pltpu.sync_copy(acc.at[pl.ds(s * ROWS_PER_S, ROWS_PER_S)],
                        out_hbm.at[c].at[pl.ds(s * ROWS_PER_S, ROWS_PER_S)])

    return k(p, src3, dst3, zeros)


def _proj1_kernel(x_ref, w_ref, o_ref):
    o_ref[0:N, :] = jnp.dot(x_ref[...], w_ref[...].T,
                            preferred_element_type=jnp.float32)
    o_ref[N:NP, :] = jnp.zeros((NP - N, HP), jnp.float32)


def _proj1(x, w_rel1p):
    return pl.pallas_call(
        _proj1_kernel,
        out_shape=jax.ShapeDtypeStruct((NP, HP), jnp.float32),
    )(x, w_rel1p)


def _combine_kernel(nrows, parts_ref, hprev_ref, wroot_ref, b_ref, wnext_ref,
                    h_ref, p_ref):
    root = jnp.dot(hprev_ref[...], wroot_ref[...].T,
                   preferred_element_type=jnp.float32)
    agg = parts_ref[0, 0:nrows, :] + parts_ref[1, 0:nrows, :]
    h_ref[0:nrows, :] = jax.nn.relu(agg + root + b_ref[...])
    if nrows < NP:
        h_ref[nrows:NP, :] = jnp.zeros((NP - nrows, HP), jnp.float32)
    p_ref[...] = jnp.dot(h_ref[...], wnext_ref[...].T,
                         preferred_element_type=jnp.float32)


def _combine(parts, hprev, wrootp, bp, wnextp):
    nrows = hprev.shape[0]
    return pl.pallas_call(
        functools.partial(_combine_kernel, nrows),
        out_shape=(jax.ShapeDtypeStruct((NP, HP), jnp.float32),
                   jax.ShapeDtypeStruct((NP, HP), jnp.float32)),
    )(parts, hprev, wrootp, bp, wnextp)


def _final_kernel(parts_ref, h2_ref, wroot_ref, b_ref, linw_ref, linb_ref,
                  o_ref):
    root = jnp.dot(h2_ref[...], wroot_ref[...].T,
                   preferred_element_type=jnp.float32)
    h3 = jax.nn.relu(parts_ref[0] + parts_ref[1] + root + b_ref[...])
    hv = h3[0:N, :]
    max_p = jnp.max(hv, axis=0)
    mean_p = jnp.sum(hv, axis=0) * (1.0 / N)
    pooled = jnp.concatenate([max_p, mean_p])[None, :]
    o_ref[...] = jnp.dot(pooled, linw_ref[...].T,
                         preferred_element_type=jnp.float32) + linb_ref[...]


def _final(parts, h2, wroot3p, b3p, lin_wp, lin_bp):
    return pl.pallas_call(
        _final_kernel,
        out_shape=jax.ShapeDtypeStruct((1, OUT), jnp.float32),
    )(parts, h2, wroot3p, b3p, lin_wp, lin_bp)


def _pad_w(w):
    return jnp.pad(w, ((0, HP - H), (0, 0)))


def _pad_w_sq(w):
    return jnp.pad(w, ((0, HP - H), (0, HP - H)))


def kernel(x, edge_index, W_rel1, W_root1, b1, W_rel2, W_root2, b2,
           W_rel3, W_root3, b3, lin_W, lin_b):
    w_rel1p = _pad_w(W_rel1)
    w_root1p = _pad_w(W_root1)
    w_rel2p = _pad_w_sq(W_rel2)
    w_root2p = _pad_w_sq(W_root2)
    w_rel3p = _pad_w_sq(W_rel3)
    w_root3p = _pad_w_sq(W_root3)
    b1p = jnp.pad(b1, (0, HP - H))[None, :]
    b2p = jnp.pad(b2, (0, HP - H))[None, :]
    b3p = jnp.pad(b3, (0, HP - H))[None, :]
    lin_wp = jnp.zeros((OUT, 2 * HP), jnp.float32)
    lin_wp = lin_wp.at[:, 0:H].set(lin_W[:, 0:H])
    lin_wp = lin_wp.at[:, HP:HP + H].set(lin_W[:, H:2 * H])
    lin_bp = lin_b[None, :]

    def slabs(idx):
        a = idx[:E0].reshape(NS, NCHUNK, CHUNK)
        b = jnp.pad(idx[E0:], (0, E1 - (E - E0)), constant_values=NP - 1)
        b = b.reshape(NS, G1 * GROUP, CHUNK)
        b = jnp.pad(b, ((0, 0), (0, NCHUNK - G1 * GROUP), (0, 0)),
                    constant_values=NP - 1)
        return jnp.concatenate([a, b], axis=0)

    src3 = slabs(edge_index[0])
    dst3 = slabs(edge_index[1])
    zeros = jnp.zeros((NP, HP), jnp.float32)

    p1 = _proj1(x, w_rel1p)
    parts1 = _sc_segment_sum(p1, src3, dst3, zeros)
    h1, p2 = _combine(parts1, x, w_root1p, b1p, w_rel2p)
    parts2 = _sc_segment_sum(p2, src3, dst3, zeros)
    h2, p3 = _combine(parts2, h1, w_root2p, b2p, w_rel3p)
    parts3 = _sc_segment_sum(p3, src3, dst3, zeros)
    return _final(parts3, h2, w_root3p, b3p, lin_wp, lin_bp)
